# baseline (speedup 1.0000x reference)
import jax
import jax.numpy as jnp
from jax import lax
from jax.experimental import pallas as pl
from jax.experimental.pallas import tpu as pltpu
from jax.experimental.pallas import tpu_sc as plsc

M, D, B = 100000, 128, 4096
NC, NS, L = 2, 16, 16
NW = NC * NS
BPW = B // NW
BPC = B // NC
WPT = B // NS
GPT = WPT // L
NV = B // L
NVC = BPC // L
SHIFT = 13
MASKJ = (1 << SHIFT) - 1

_DNUMS = lax.GatherDimensionNumbers(
    offset_dims=(), collapsed_slice_dims=(0,), start_index_map=(0,))


def _body(mem_h, widx_h, sval_h, sidx_h, neg1_h, out_h,
          owner_v, wslice_v, stage_v, stage_full_v, sidx_sc_v, ovr_v,
          sidx_v, rows_v, shared_stage, shared_ovr, sem, sem2):
    c = lax.axis_index("c")
    s = lax.axis_index("s")
    wid = c * NS + s
    base = wid * BPW

    pltpu.sync_copy(sidx_h.at[pl.ds(base, BPW)], sidx_v)
    rows_dma = pltpu.async_copy(mem_h.at[sidx_v], rows_v, sem)

    @pl.when(s == 0)
    def _owner_init():
        pltpu.async_copy(neg1_h, owner_v, sem2)
        pltpu.sync_copy(sidx_h.at[pl.ds(c * BPC, BPC)], sidx_sc_v)

    pltpu.sync_copy(widx_h.at[pl.ds(s * WPT, WPT)], wslice_v)
    iota = lax.iota(jnp.int32, L)
    nxt_lane = jnp.minimum(iota + 1, L - 1)
    last_lane = iota == (L - 1)
    jbase = s * WPT

    @plsc.parallel_loop(0, GPT, unroll=8)
    def _stage(g):
        w16 = wslice_v[pl.ds(g * L, L)]
        key = (w16 << SHIFT) | (jbase + g * L + iota)
        key = plsc.sort_key_val(key, key)[0]
        rows = lax.shift_right_logical(key, SHIFT)
        nxt = lax.gather(rows, nxt_lane[:, None], _DNUMS, (1,),
                         mode=lax.GatherScatterMode.PROMISE_IN_BOUNDS)
        keep = (rows != nxt) | last_lane
        stage_v[pl.ds(g * L, L)] = jnp.where(keep, key, -1)
    pltpu.sync_copy(stage_v, shared_stage.at[pl.ds(s * WPT, WPT)])
    plsc.subcore_barrier()

    @pl.when(s == 0)
    def _owner_work():
        pltpu.sync_copy(shared_stage, stage_full_v)
        pltpu.make_async_copy(neg1_h, owner_v, sem2).wait()

        @plsc.parallel_loop(0, NV, unroll=16)
        def _scat(v):
            key = stage_full_v[pl.ds(v * L, L)]
            rows = lax.shift_right_logical(key, SHIFT)
            js = key & MASKJ
            plsc.store_scatter(owner_v, [rows], js, mask=key >= 0)

        @plsc.parallel_loop(0, NVC, unroll=16)
        def _lut(v):
            sv = sidx_sc_v[pl.ds(v * L, L)]
            ovr_v[pl.ds(v * L, L)] = plsc.load_gather(owner_v, [sv])
        pltpu.sync_copy(ovr_v, shared_ovr)

    plsc.subcore_barrier()
    rows_dma.wait()
    pltpu.sync_copy(shared_ovr.at[pl.ds(s * BPW, BPW)], sidx_v)

    def patch(g, carry):
        ovr16 = sidx_v[pl.ds(g * L, L)]
        for k in range(L):
            o = ovr16[k]

            @pl.when(o >= 0)
            def _():
                pltpu.sync_copy(sval_h.at[o], rows_v.at[g * L + k])

        return carry

    lax.fori_loop(0, BPW // L, patch, 0)
    pltpu.sync_copy(rows_v, out_h.at[pl.ds(base, BPW)])


@jax.jit
def kernel(mem, write_idx, s_val, sample_idx):
    widx = write_idx.astype(jnp.int32)
    sidx = sample_idx.astype(jnp.int32)
    neg1 = jnp.full((M,), -1, jnp.int32)
    mesh = plsc.VectorSubcoreMesh(core_axis_name="c", subcore_axis_name="s")
    f = pl.kernel(
        _body,
        out_type=jax.ShapeDtypeStruct((B, D), jnp.float32),
        mesh=mesh,
        compiler_params=pltpu.CompilerParams(needs_layout_passes=False),
        scratch_types=[
            pltpu.VMEM((M,), jnp.int32),
            pltpu.VMEM((WPT,), jnp.int32),
            pltpu.VMEM((WPT,), jnp.int32),
            pltpu.VMEM((B,), jnp.int32),
            pltpu.VMEM((BPC,), jnp.int32),
            pltpu.VMEM((BPC,), jnp.int32),
            pltpu.VMEM((BPW,), jnp.int32),
            pltpu.VMEM((BPW, D), jnp.float32),
            pltpu.VMEM_SHARED((B,), jnp.int32),
            pltpu.VMEM_SHARED((BPC,), jnp.int32),
            pltpu.SemaphoreType.DMA,
            pltpu.SemaphoreType.DMA,
        ],
    )
    return f(mem, widx, s_val, sidx, neg1)


# speedup vs baseline: 1.0088x; 1.0088x over previous
import jax
import jax.numpy as jnp
from jax import lax
from jax.experimental import pallas as pl
from jax.experimental.pallas import tpu as pltpu
from jax.experimental.pallas import tpu_sc as plsc

M, D, B = 100000, 128, 4096
NC, NS, L = 2, 16, 16
NW = NC * NS
BPW = B // NW
BPC = B // NC
WPT = B // NS
GPT = WPT // L
NV = B // L
NVC = BPC // L
SHIFT = 13
MASKJ = (1 << SHIFT) - 1

_DNUMS = lax.GatherDimensionNumbers(
    offset_dims=(), collapsed_slice_dims=(0,), start_index_map=(0,))


def _body(mem_h, widx_h, sval_h, sidx_h, neg1_h, out_h,
          owner_v, wslice_v, stage_v, stage_full_v, sidx_sc_v, ovr_v,
          sidx_v, rows_v, shared_stage, shared_ovr, sem, sem2):
    c = lax.axis_index("c")
    s = lax.axis_index("s")
    wid = c * NS + s
    base = wid * BPW

    pltpu.sync_copy(sidx_h.at[pl.ds(base, BPW)], sidx_v)
    rows_dma = pltpu.async_copy(mem_h.at[sidx_v], rows_v, sem)

    @pl.when(s == 0)
    def _owner_init():
        pltpu.async_copy(neg1_h, owner_v, sem2)
        pltpu.sync_copy(sidx_h.at[pl.ds(c * BPC, BPC)], sidx_sc_v)

    pltpu.sync_copy(widx_h.at[pl.ds(s * WPT, WPT)], wslice_v)
    iota = lax.iota(jnp.int32, L)
    nxt_lane = jnp.minimum(iota + 1, L - 1)
    last_lane = iota == (L - 1)
    jbase = s * WPT

    @plsc.parallel_loop(0, GPT, unroll=4)
    def _stage(g):
        w16 = wslice_v[pl.ds(g * L, L)]
        key = (w16 << SHIFT) | (jbase + g * L + iota)
        key = plsc.sort_key_val(key, key)[0]
        rows = lax.shift_right_logical(key, SHIFT)
        nxt = lax.gather(rows, nxt_lane[:, None], _DNUMS, (1,),
                         mode=lax.GatherScatterMode.PROMISE_IN_BOUNDS)
        keep = (rows != nxt) | last_lane
        stage_v[pl.ds(g * L, L)] = jnp.where(keep, key, -1)
    pltpu.sync_copy(stage_v, shared_stage.at[pl.ds(s * WPT, WPT)])
    plsc.subcore_barrier()

    @pl.when(s == 0)
    def _owner_work():
        pltpu.sync_copy(shared_stage, stage_full_v)
        pltpu.make_async_copy(neg1_h, owner_v, sem2).wait()

        @plsc.parallel_loop(0, NV, unroll=8)
        def _scat(v):
            key = stage_full_v[pl.ds(v * L, L)]
            rows = lax.shift_right_logical(key, SHIFT)
            js = key & MASKJ
            plsc.store_scatter(owner_v, [rows], js, mask=key >= 0)

        @plsc.parallel_loop(0, NVC, unroll=8)
        def _lut(v):
            sv = sidx_sc_v[pl.ds(v * L, L)]
            ovr_v[pl.ds(v * L, L)] = plsc.load_gather(owner_v, [sv])
        pltpu.sync_copy(ovr_v, shared_ovr)

    plsc.subcore_barrier()
    rows_dma.wait()
    pltpu.sync_copy(shared_ovr.at[pl.ds(s * BPW, BPW)], sidx_v)

    def patch(g, carry):
        ovr16 = sidx_v[pl.ds(g * L, L)]
        for k in range(L):
            o = ovr16[k]

            @pl.when(o >= 0)
            def _():
                pltpu.sync_copy(sval_h.at[o], rows_v.at[g * L + k])

        return carry

    lax.fori_loop(0, BPW // L, patch, 0)
    pltpu.sync_copy(rows_v, out_h.at[pl.ds(base, BPW)])


@jax.jit
def kernel(mem, write_idx, s_val, sample_idx):
    widx = write_idx.astype(jnp.int32)
    sidx = sample_idx.astype(jnp.int32)
    neg1 = jnp.full((M,), -1, jnp.int32)
    mesh = plsc.VectorSubcoreMesh(core_axis_name="c", subcore_axis_name="s")
    f = pl.kernel(
        _body,
        out_type=jax.ShapeDtypeStruct((B, D), jnp.float32),
        mesh=mesh,
        compiler_params=pltpu.CompilerParams(needs_layout_passes=False, disable_bounds_checks=True, disable_semaphore_checks=True, skip_device_barrier=True),
        scratch_types=[
            pltpu.VMEM((M,), jnp.int32),
            pltpu.VMEM((WPT,), jnp.int32),
            pltpu.VMEM((WPT,), jnp.int32),
            pltpu.VMEM((B,), jnp.int32),
            pltpu.VMEM((BPC,), jnp.int32),
            pltpu.VMEM((BPC,), jnp.int32),
            pltpu.VMEM((BPW,), jnp.int32),
            pltpu.VMEM((BPW, D), jnp.float32),
            pltpu.VMEM_SHARED((B,), jnp.int32),
            pltpu.VMEM_SHARED((BPC,), jnp.int32),
            pltpu.SemaphoreType.DMA,
            pltpu.SemaphoreType.DMA,
        ],
    )
    return f(mem, widx, s_val, sidx, neg1)


# speedup vs baseline: 1.1645x; 1.1544x over previous
import jax
import jax.numpy as jnp
from jax import lax
from jax.experimental import pallas as pl
from jax.experimental.pallas import tpu as pltpu
from jax.experimental.pallas import tpu_sc as plsc

M, D, B = 100000, 128, 4096
NC, NS, L = 2, 16, 16
NW = NC * NS
BPW = B // NW
BPC = B // NC
WPT = B // NS
GPT = WPT // L
NV = B // L
NVC = BPC // L
SHIFT = 13
MASKJ = (1 << SHIFT) - 1

_DNUMS = lax.GatherDimensionNumbers(
    offset_dims=(), collapsed_slice_dims=(0,), start_index_map=(0,))


def _body(mem_h, widx_h, sval_h, sidx_h, neg1_h, out_h,
          owner_v, wslice_v, stage_v, stage_full_v, sidx_sc_v, ovr_v,
          sidx_v, rows_v, shared_stage, shared_ovr, sem, sem2, sem3, sem4):
    c = lax.axis_index("c")
    s = lax.axis_index("s")
    wid = c * NS + s
    base = wid * BPW

    pltpu.sync_copy(sidx_h.at[pl.ds(base, BPW)], sidx_v)
    rows_dma = pltpu.async_copy(mem_h.at[sidx_v], rows_v, sem)

    @pl.when(s == 0)
    def _owner_init():
        pltpu.async_copy(neg1_h, owner_v, sem2)
        pltpu.sync_copy(sidx_h.at[pl.ds(c * BPC, BPC)], sidx_sc_v)

    pltpu.sync_copy(widx_h.at[pl.ds(s * WPT, WPT)], wslice_v)
    iota = lax.iota(jnp.int32, L)
    nxt_lane = jnp.minimum(iota + 1, L - 1)
    last_lane = iota == (L - 1)
    jbase = s * WPT

    @plsc.parallel_loop(0, GPT, unroll=4)
    def _stage(g):
        w16 = wslice_v[pl.ds(g * L, L)]
        key = (w16 << SHIFT) | (jbase + g * L + iota)
        key = plsc.sort_key_val(key, key)[0]
        rows = lax.shift_right_logical(key, SHIFT)
        nxt = lax.gather(rows, nxt_lane[:, None], _DNUMS, (1,),
                         mode=lax.GatherScatterMode.PROMISE_IN_BOUNDS)
        keep = (rows != nxt) | last_lane
        stage_v[pl.ds(g * L, L)] = jnp.where(keep, key, -1)
    pltpu.sync_copy(stage_v, shared_stage.at[pl.ds(s * WPT, WPT)])
    plsc.subcore_barrier()

    @pl.when(s != 0)
    def _early_out():
        rows_dma.wait()
        pltpu.sync_copy(rows_v, out_h.at[pl.ds(base, BPW)])

    @pl.when(s == 0)
    def _owner_work():
        pltpu.sync_copy(shared_stage, stage_full_v)
        rows_dma.wait()
        out_dma = pltpu.async_copy(rows_v, out_h.at[pl.ds(base, BPW)], sem4)
        pltpu.make_async_copy(neg1_h, owner_v, sem2).wait()

        @plsc.parallel_loop(0, NV, unroll=8)
        def _scat(v):
            key = stage_full_v[pl.ds(v * L, L)]
            rows = lax.shift_right_logical(key, SHIFT)
            js = key & MASKJ
            plsc.store_scatter(owner_v, [rows], js, mask=key >= 0)

        @plsc.parallel_loop(0, NVC, unroll=8)
        def _lut(v):
            sv = sidx_sc_v[pl.ds(v * L, L)]
            ovr_v[pl.ds(v * L, L)] = plsc.load_gather(owner_v, [sv])
        pltpu.sync_copy(ovr_v, shared_ovr)
        out_dma.wait()

    plsc.subcore_barrier()

    pltpu.sync_copy(shared_ovr.at[pl.ds(s * BPW, BPW)], sidx_v)

    def patch1(g, cnt):
        ovr16 = sidx_v[pl.ds(g * L, L)]
        for k in range(L):
            o = ovr16[k]

            @pl.when(o >= 0)
            def _():
                pltpu.async_copy(sval_h.at[o], rows_v.at[g * L + k], sem3)

        return cnt + plsc.all_reduce_population_count(ovr16 >= 0)[0]

    cnt = lax.fori_loop(0, BPW // L, patch1, jnp.int32(0))

    def drain(i, carry):
        pltpu.make_async_copy(sval_h.at[0], rows_v.at[0], sem3).wait()
        return carry

    lax.fori_loop(0, cnt, drain, 0)

    def patch2(g, carry):
        ovr16 = sidx_v[pl.ds(g * L, L)]
        for k in range(L):
            o = ovr16[k]

            @pl.when(o >= 0)
            def _():
                pltpu.async_copy(rows_v.at[g * L + k],
                                 out_h.at[base + g * L + k], sem3)

        return carry

    lax.fori_loop(0, BPW // L, patch2, 0)
    lax.fori_loop(0, cnt, drain, 0)


@jax.jit
def kernel(mem, write_idx, s_val, sample_idx):
    widx = write_idx.astype(jnp.int32)
    sidx = sample_idx.astype(jnp.int32)
    neg1 = jnp.full((M,), -1, jnp.int32)
    mesh = plsc.VectorSubcoreMesh(core_axis_name="c", subcore_axis_name="s")
    f = pl.kernel(
        _body,
        out_type=jax.ShapeDtypeStruct((B, D), jnp.float32),
        mesh=mesh,
        compiler_params=pltpu.CompilerParams(needs_layout_passes=False),
        scratch_types=[
            pltpu.VMEM((M,), jnp.int32),
            pltpu.VMEM((WPT,), jnp.int32),
            pltpu.VMEM((WPT,), jnp.int32),
            pltpu.VMEM((B,), jnp.int32),
            pltpu.VMEM((BPC,), jnp.int32),
            pltpu.VMEM((BPC,), jnp.int32),
            pltpu.VMEM((BPW,), jnp.int32),
            pltpu.VMEM((BPW, D), jnp.float32),
            pltpu.VMEM_SHARED((B,), jnp.int32),
            pltpu.VMEM_SHARED((BPC,), jnp.int32),
            pltpu.SemaphoreType.DMA,
            pltpu.SemaphoreType.DMA,
            pltpu.SemaphoreType.DMA,
            pltpu.SemaphoreType.DMA,
        ],
    )
    return f(mem, widx, s_val, sidx, neg1)
